# baseline (speedup 1.0000x reference)
import functools

import jax
import jax.numpy as jnp
from jax import lax
from jax.experimental import pallas as pl
from jax.experimental.pallas import tpu as pltpu
from jax.experimental.pallas import tpu_sc as plsc

B = 16384
EMBED_DIM = 16
NV = 1000000
NC = 2
NS = 16
NW = NC * NS
B_PER_W = B // NW
KBUF = 24


def _sc_gather_body(uidx_hbm, iidx_hbm, utabT_hbm, itabT_hbm,
                    uout_hbm, iout_hbm,
                    idx_v, ring, rows_v, sems):
    wid = lax.axis_index("s") * NC + lax.axis_index("c")
    base = wid * B_PER_W
    lane = lax.iota(jnp.int32, EMBED_DIM)

    n_waves = B_PER_W // 16

    def gather_one(idx_hbm, tabT_hbm, out_hbm):
        pltpu.sync_copy(idx_hbm.at[pl.ds(base, B_PER_W)], idx_v)

        def fire(i, slot):
            c0 = pl.multiple_of((i // 128) * 128, 128)
            pltpu.async_copy(tabT_hbm.at[:, pl.ds(c0, 128)],
                             ring.at[slot], sems.at[slot])

        iv0 = idx_v[pl.ds(0, 16)]
        iv1 = idx_v[pl.ds(16, 16)]
        for k in range(16):
            fire(iv0[k], k)
        for k in range(8):
            fire(iv1[k], 16 + k)

        def body(q, carry):
            iv = idx_v[pl.ds(q * 16, 16)]
            qn1 = jnp.minimum(q + 1, n_waves - 1)
            ivn1 = idx_v[pl.ds(qn1 * 16, 16)]
            qn2 = jnp.minimum(q + 2, n_waves - 1)
            ivn2 = idx_v[pl.ds(qn2 * 16, 16)]
            for k in range(16):
                p = q * 16 + k
                slot = lax.rem(p, KBUF)
                pltpu.make_async_copy(
                    tabT_hbm.at[:, pl.ds(0, 128)], ring.at[slot], sems.at[slot]
                ).wait()
                di = lax.rem(iv[k], 128)
                vals = plsc.load_gather(ring.at[slot], [lane, di + 0 * lane])
                plsc.store_scatter(
                    rows_v,
                    [jnp.full((EMBED_DIM,), 0, jnp.int32) + p, lane],
                    vals)
                nidx = ivn1[k + 8] if k < 8 else ivn2[k - 8]

                @pl.when(p < B_PER_W - KBUF)
                def _refill():
                    fire(nidx, slot)

            return carry

        lax.fori_loop(0, n_waves, body, 0)
        pltpu.sync_copy(rows_v, out_hbm.at[pl.ds(base, B_PER_W)])

    gather_one(uidx_hbm, utabT_hbm, uout_hbm)
    gather_one(iidx_hbm, itabT_hbm, iout_hbm)


@jax.jit
def _sc_gather(user_idx, item_idx, utabT, itabT):
    mesh = plsc.VectorSubcoreMesh(core_axis_name="c", subcore_axis_name="s")
    f = pl.kernel(
        _sc_gather_body,
        out_type=(
            jax.ShapeDtypeStruct((B, EMBED_DIM), jnp.float32),
            jax.ShapeDtypeStruct((B, EMBED_DIM), jnp.float32),
        ),
        mesh=mesh,
        scratch_types=[
            pltpu.VMEM((B_PER_W,), jnp.int32),
            pltpu.VMEM((KBUF, EMBED_DIM, 128), jnp.float32),
            pltpu.VMEM((B_PER_W, EMBED_DIM), jnp.float32),
            pltpu.SemaphoreType.DMA((KBUF,)),
        ],
        compiler_params=pltpu.CompilerParams(needs_layout_passes=False),
    )
    return f(user_idx, item_idx, utabT, itabT)


def _mlp_body(u_ref, i_ref, dc_ref, w1u_ref, w1i_ref, w1dc_ref, b1_ref,
              w2_ref, b2_ref, wh_ref, bh_ref, out_ref):
    h = (jnp.dot(u_ref[...], w1u_ref[...], preferred_element_type=jnp.float32)
         + jnp.dot(i_ref[...], w1i_ref[...], preferred_element_type=jnp.float32)
         + jnp.dot(dc_ref[...], w1dc_ref[...], preferred_element_type=jnp.float32)
         + b1_ref[...])
    h = jnp.maximum(h, 0.0)
    h = jnp.dot(h, w2_ref[...], preferred_element_type=jnp.float32) + b2_ref[...]
    h = jnp.maximum(h, 0.0)
    out_ref[...] = jax.nn.sigmoid(
        jnp.dot(h, wh_ref[...], preferred_element_type=jnp.float32) + bh_ref[...])


@jax.jit
def _tc_mlp(u_emb, i_emb, dc, w1u, w1i, w1dc, b1, w2, b2, wh, bh):
    BB = 2048
    grid = (B // BB,)
    dcdim = dc.shape[1]
    return pl.pallas_call(
        _mlp_body,
        grid=grid,
        in_specs=[
            pl.BlockSpec((BB, EMBED_DIM), lambda i: (i, 0)),
            pl.BlockSpec((BB, EMBED_DIM), lambda i: (i, 0)),
            pl.BlockSpec((BB, dcdim), lambda i: (i, 0)),
            pl.BlockSpec(w1u.shape, lambda i: (0, 0)),
            pl.BlockSpec(w1i.shape, lambda i: (0, 0)),
            pl.BlockSpec(w1dc.shape, lambda i: (0, 0)),
            pl.BlockSpec(b1.shape, lambda i: (0, 0)),
            pl.BlockSpec(w2.shape, lambda i: (0, 0)),
            pl.BlockSpec(b2.shape, lambda i: (0, 0)),
            pl.BlockSpec(wh.shape, lambda i: (0, 0)),
            pl.BlockSpec(bh.shape, lambda i: (0, 0)),
        ],
        out_specs=pl.BlockSpec((BB, 2), lambda i: (i, 0)),
        out_shape=jax.ShapeDtypeStruct((B, 2), jnp.float32),
    )(u_emb, i_emb, dc, w1u, w1i, w1dc, b1, w2, b2, wh, bh)


def kernel(user_idx, item_idx, dense_feats, comment_emb, user_table, item_table,
           W1, b1, W2, b2, ctr_w, ctr_b, cvr_w, cvr_b):
    user_idx = user_idx.astype(jnp.int32)
    item_idx = item_idx.astype(jnp.int32)
    u_emb, i_emb = _sc_gather(user_idx, item_idx, user_table.T, item_table.T)
    dc = jnp.concatenate([dense_feats, comment_emb], axis=-1)
    w1u = W1[:EMBED_DIM]
    w1i = W1[EMBED_DIM:2 * EMBED_DIM]
    w1dc = W1[2 * EMBED_DIM:]
    wh = jnp.concatenate([ctr_w, cvr_w], axis=1)
    bh = jnp.stack([ctr_b[0], cvr_b[0]])[None, :]
    out = _tc_mlp(u_emb, i_emb, dc, w1u, w1i, w1dc, b1[None, :], W2,
                  b2[None, :], wh, bh)
    return out[:, 0], out[:, 1]


# speedup vs baseline: 1.0248x; 1.0248x over previous
import functools

import jax
import jax.numpy as jnp
from jax import lax
from jax.experimental import pallas as pl
from jax.experimental.pallas import tpu as pltpu
from jax.experimental.pallas import tpu_sc as plsc

B = 16384
EMBED_DIM = 16
NV = 1000000
NC = 2
NS = 16
NW = NC * NS
B_PER_W = B // NW
KBUF = 24


def _sc_gather_body(uidx_hbm, iidx_hbm, utabT_hbm, itabT_hbm,
                    uout_hbm, iout_hbm,
                    idx_v, ring, rows_v, sems):
    wid = lax.axis_index("s") * NC + lax.axis_index("c")
    base = wid * B_PER_W
    lane = lax.iota(jnp.int32, EMBED_DIM)

    n_waves = B_PER_W // 16

    def gather_one(idx_hbm, tabT_hbm, out_hbm):
        pltpu.sync_copy(idx_hbm.at[pl.ds(base, B_PER_W)], idx_v)

        def fire(i, slot):
            c0 = pl.multiple_of((i // 128) * 128, 128)
            pltpu.async_copy(tabT_hbm.at[:, pl.ds(c0, 128)],
                             ring.at[slot], sems.at[slot])

        iv0 = idx_v[pl.ds(0, 16)]
        iv1 = idx_v[pl.ds(16, 16)]
        for k in range(16):
            fire(iv0[k], k)
        for k in range(8):
            fire(iv1[k], 16 + k)

        def body(q, carry):
            iv = idx_v[pl.ds(q * 16, 16)]
            qn1 = jnp.minimum(q + 1, n_waves - 1)
            ivn1 = idx_v[pl.ds(qn1 * 16, 16)]
            qn2 = jnp.minimum(q + 2, n_waves - 1)
            ivn2 = idx_v[pl.ds(qn2 * 16, 16)]
            for k in range(16):
                p = q * 16 + k
                slot = lax.rem(p, KBUF)
                pltpu.make_async_copy(
                    tabT_hbm.at[:, pl.ds(0, 128)], ring.at[slot], sems.at[slot]
                ).wait()
                di = lax.rem(iv[k], 128)
                vals = plsc.load_gather(ring.at[slot], [lane, di + 0 * lane])
                plsc.store_scatter(
                    rows_v,
                    [jnp.full((EMBED_DIM,), 0, jnp.int32) + p, lane],
                    vals)
                nidx = ivn1[k + 8] if k < 8 else ivn2[k - 8]

                @pl.when(p < B_PER_W - KBUF)
                def _refill():
                    fire(nidx, slot)

            return carry

        lax.fori_loop(0, n_waves, body, 0)
        pltpu.sync_copy(rows_v, out_hbm.at[pl.ds(base, B_PER_W)])

    gather_one(uidx_hbm, utabT_hbm, uout_hbm)
    gather_one(iidx_hbm, itabT_hbm, iout_hbm)


@jax.jit
def _sc_gather(user_idx, item_idx, utabT, itabT):
    mesh = plsc.VectorSubcoreMesh(core_axis_name="c", subcore_axis_name="s")
    f = pl.kernel(
        _sc_gather_body,
        out_type=(
            jax.ShapeDtypeStruct((B, EMBED_DIM), jnp.float32),
            jax.ShapeDtypeStruct((B, EMBED_DIM), jnp.float32),
        ),
        mesh=mesh,
        scratch_types=[
            pltpu.VMEM((B_PER_W,), jnp.int32),
            pltpu.VMEM((KBUF, EMBED_DIM, 128), jnp.float32),
            pltpu.VMEM((B_PER_W, EMBED_DIM), jnp.float32),
            pltpu.SemaphoreType.DMA((KBUF,)),
        ],
        compiler_params=pltpu.CompilerParams(needs_layout_passes=False),
    )
    return f(user_idx, item_idx, utabT, itabT)


def _mlp_body(u_ref, i_ref, d_ref, c_ref, w1u_ref, w1i_ref, w1d_ref, w1c_ref,
              b1_ref, w2_ref, b2_ref, wh_ref, bh_ref, ctr_ref, cvr_ref):
    h = (jnp.dot(u_ref[...], w1u_ref[...], preferred_element_type=jnp.float32)
         + jnp.dot(i_ref[...], w1i_ref[...], preferred_element_type=jnp.float32)
         + jnp.dot(d_ref[...], w1d_ref[...], preferred_element_type=jnp.float32)
         + jnp.dot(c_ref[...], w1c_ref[...], preferred_element_type=jnp.float32)
         + b1_ref[...])
    h = jnp.maximum(h, 0.0)
    h = jnp.dot(h, w2_ref[...], preferred_element_type=jnp.float32) + b2_ref[...]
    h = jnp.maximum(h, 0.0)
    hh = jax.nn.sigmoid(
        jnp.dot(h, wh_ref[...], preferred_element_type=jnp.float32) + bh_ref[...])
    ctr_ref[...] = hh[:, 0]
    cvr_ref[...] = hh[:, 1]


@jax.jit
def _tc_mlp(u_emb, i_emb, d, c, w1u, w1i, w1d, w1c, b1, w2, b2, wh, bh):
    BB = 2048
    grid = (B // BB,)
    return pl.pallas_call(
        _mlp_body,
        grid=grid,
        in_specs=[
            pl.BlockSpec((BB, EMBED_DIM), lambda i: (i, 0)),
            pl.BlockSpec((BB, EMBED_DIM), lambda i: (i, 0)),
            pl.BlockSpec((BB, 3), lambda i: (i, 0)),
            pl.BlockSpec((BB, 32), lambda i: (i, 0)),
            pl.BlockSpec(w1u.shape, lambda i: (0, 0)),
            pl.BlockSpec(w1i.shape, lambda i: (0, 0)),
            pl.BlockSpec(w1d.shape, lambda i: (0, 0)),
            pl.BlockSpec(w1c.shape, lambda i: (0, 0)),
            pl.BlockSpec(b1.shape, lambda i: (0, 0)),
            pl.BlockSpec(w2.shape, lambda i: (0, 0)),
            pl.BlockSpec(b2.shape, lambda i: (0, 0)),
            pl.BlockSpec(wh.shape, lambda i: (0, 0)),
            pl.BlockSpec(bh.shape, lambda i: (0, 0)),
        ],
        out_specs=[pl.BlockSpec((BB,), lambda i: (i,)),
                   pl.BlockSpec((BB,), lambda i: (i,))],
        out_shape=[jax.ShapeDtypeStruct((B,), jnp.float32),
                   jax.ShapeDtypeStruct((B,), jnp.float32)],
    )(u_emb, i_emb, d, c, w1u, w1i, w1d, w1c, b1, w2, b2, wh, bh)


def kernel(user_idx, item_idx, dense_feats, comment_emb, user_table, item_table,
           W1, b1, W2, b2, ctr_w, ctr_b, cvr_w, cvr_b):
    user_idx = user_idx.astype(jnp.int32)
    item_idx = item_idx.astype(jnp.int32)
    u_emb, i_emb = _sc_gather(user_idx, item_idx, user_table.T, item_table.T)
    w1u = W1[:EMBED_DIM]
    w1i = W1[EMBED_DIM:2 * EMBED_DIM]
    w1d = W1[2 * EMBED_DIM:2 * EMBED_DIM + 3]
    w1c = W1[2 * EMBED_DIM + 3:]
    wh = jnp.concatenate([ctr_w, cvr_w], axis=1)
    bh = jnp.stack([ctr_b[0], cvr_b[0]])[None, :]
    ctr, cvr = _tc_mlp(u_emb, i_emb, dense_feats, comment_emb, w1u, w1i, w1d,
                       w1c, b1[None, :], W2, b2[None, :], wh, bh)
    return ctr, cvr
